# baseline (speedup 1.0000x reference)
import functools

import jax
import jax.numpy as jnp
from jax import lax
from jax.experimental import pallas as pl
from jax.experimental.pallas import tpu as pltpu
from jax.experimental.pallas import tpu_sc as plsc

NC, NS, LANES = 2, 16, 16
NW = NC * NS

N_SEG = 10000
R_DIM = 1000
D = 128
DH = D // 2
E = 320000
EB = 128
N_PAD = 10240
DUMP = N_PAD - 1
NB_TOTAL = 2560
E_PAD = NB_TOTAL * EB
RPT = NB_TOTAL // NW
KG = 1
NT = RPT // (2 * KG)
ROWS_ACC = N_PAD // NS
DEG_W = 16
NROW_BLK = 1000
GRID_N = N_SEG // NROW_BLK


def _sc_deg_body(src_hbm, zeros_hbm, out_hbm, idx_v, buf_v, sem, acc):
    c = lax.axis_index("c")
    s = lax.axis_index("s")
    wid = c * NS + s

    pltpu.sync_copy(zeros_hbm, acc.at[pl.ds(s * ROWS_ACC, ROWS_ACC)])

    o = jnp.ones((LANES,), jnp.float32)

    def fill_one(i, _):
        buf_v[i, :] = o
        return 0

    lax.fori_loop(0, EB, fill_one, 0)
    pltpu.sync_copy(src_hbm.at[pl.ds(wid * RPT, RPT)], idx_v)
    plsc.subcore_barrier()

    def step(t, _):
        descs = [
            pltpu.async_copy(buf_v, acc.at[idx_v.at[t * 5 + i]], sem,
                             add=True)
            for i in range(5)
        ]
        for d in descs:
            d.wait()
        return 0

    lax.fori_loop(0, RPT // 5, step, 0)
    plsc.subcore_barrier()
    pltpu.sync_copy(acc.at[pl.ds(s * ROWS_ACC, ROWS_ACC)],
                    out_hbm.at[c, pl.ds(s * ROWS_ACC, ROWS_ACC)])


_sc_deg = pl.kernel(
    _sc_deg_body,
    out_type=jax.ShapeDtypeStruct((NC, N_PAD, DEG_W), jnp.float32),
    mesh=plsc.VectorSubcoreMesh(core_axis_name="c", subcore_axis_name="s"),
    scratch_types=[
        pltpu.VMEM((RPT, EB), jnp.int32),
        pltpu.VMEM((EB, DEG_W), jnp.float32),
        pltpu.SemaphoreType.DMA,
        pltpu.VMEM_SHARED((N_PAD, DEG_W), jnp.float32),
    ],
    compiler_params=pltpu.CompilerParams(use_tc_tiling_on_sc=False),
)


def _sc_matvec_body(g_hbm, src_hbm, dst_hbm, zeros_hbm, out_hbm,
                    isrc_v, idst_v, rows_v, gsem, ssem_a, ssem_b, acc, gst):
    c = lax.axis_index("c")
    s = lax.axis_index("s")
    wid = c * NS + s

    pltpu.sync_copy(src_hbm.at[pl.ds(wid * RPT, RPT)], isrc_v)
    pltpu.sync_copy(dst_hbm.at[pl.ds(wid * RPT, RPT)], idst_v)
    pltpu.sync_copy(zeros_hbm, acc.at[pl.ds(s * ROWS_ACC, ROWS_ACC)])
    pltpu.sync_copy(g_hbm.at[pl.ds(s * ROWS_ACC, ROWS_ACC)],
                    gst.at[pl.ds(s * ROWS_ACC, ROWS_ACC)])
    for b in range(KG):
        pltpu.async_copy(g_hbm.at[pl.ds(0, EB)], rows_v.at[KG + b],
                         ssem_b)
    plsc.subcore_barrier()

    def step(t, _):
        base = t * 2 * KG
        ga = [
            pltpu.async_copy(gst.at[isrc_v.at[base + i]],
                             rows_v.at[i], gsem)
            for i in range(KG)
        ]
        for d in ga:
            d.wait()
        sa = [
            pltpu.async_copy(rows_v.at[i], acc.at[idst_v.at[base + i]],
                             ssem_a, add=True)
            for i in range(KG)
        ]
        for b in range(KG):
            pltpu.make_async_copy(g_hbm.at[pl.ds(0, EB)],
                                  rows_v.at[KG + b], ssem_b).wait()
        gb = [
            pltpu.async_copy(gst.at[isrc_v.at[base + KG + i]],
                             rows_v.at[KG + i], gsem)
            for i in range(KG)
        ]
        for d in gb:
            d.wait()
        for d in sa:
            d.wait()
        for i in range(KG):
            pltpu.async_copy(rows_v.at[KG + i],
                             acc.at[idst_v.at[base + KG + i]],
                             ssem_b, add=True)
        return 0

    lax.fori_loop(0, NT, step, 0)
    for b in range(KG):
        pltpu.make_async_copy(g_hbm.at[pl.ds(0, EB)],
                              rows_v.at[KG + b], ssem_b).wait()
    plsc.subcore_barrier()
    pltpu.sync_copy(acc.at[pl.ds(s * ROWS_ACC, ROWS_ACC)],
                    out_hbm.at[c, pl.ds(s * ROWS_ACC, ROWS_ACC)])


_sc_matvec = pl.kernel(
    _sc_matvec_body,
    out_type=jax.ShapeDtypeStruct((NC, N_PAD, DH), jnp.float32),
    mesh=plsc.VectorSubcoreMesh(core_axis_name="c", subcore_axis_name="s"),
    scratch_types=[
        pltpu.VMEM((RPT, EB), jnp.int32),
        pltpu.VMEM((RPT, EB), jnp.int32),
        pltpu.VMEM((2 * KG, EB, DH), jnp.float32),
        pltpu.SemaphoreType.DMA,
        pltpu.SemaphoreType.DMA,
        pltpu.SemaphoreType.DMA,
        pltpu.VMEM_SHARED((N_PAD, DH), jnp.float32),
        pltpu.VMEM_SHARED((N_PAD, DH), jnp.float32),
    ],
    compiler_params=pltpu.CompilerParams(use_tc_tiling_on_sc=False),
)


_half_spec = pl.BlockSpec((NROW_BLK, DH), lambda i: (i, 0))
_row_spec = pl.BlockSpec((NROW_BLK, D), lambda i: (i, 0))
_dis_spec = pl.BlockSpec((NROW_BLK, 1), lambda i: (i, 0))


def _mm_body(s_ref, p_ref, raw_ref):
    raw_ref[...] = jnp.dot(s_ref[...], p_ref[...],
                           preferred_element_type=jnp.float32)


_t_matmul = pl.pallas_call(
    _mm_body,
    grid=(GRID_N,),
    in_specs=[
        pl.BlockSpec((NROW_BLK, R_DIM), lambda i: (i, 0)),
        pl.BlockSpec((R_DIM, D), lambda i: (0, 0)),
    ],
    out_specs=_row_spec,
    out_shape=jax.ShapeDtypeStruct((N_SEG, D), jnp.float32),
)


def _dis_g_body(p_ref, raw_ref, dis_ref, glo_ref, ghi_ref):
    i = pl.program_id(0)
    d = (p_ref[0, pl.ds(i * NROW_BLK, NROW_BLK), 0:1]
         + p_ref[1, pl.ds(i * NROW_BLK, NROW_BLK), 0:1])
    dis = jnp.where(d > 0.0, lax.rsqrt(jnp.maximum(d, 1.0)), 0.0)
    dis_ref[...] = dis
    g = raw_ref[...] * dis
    glo_ref[...] = g[:, :DH]
    ghi_ref[...] = g[:, DH:]


_t_dis_g = pl.pallas_call(
    _dis_g_body,
    grid=(GRID_N,),
    in_specs=[
        pl.BlockSpec((NC, N_PAD, DEG_W), lambda i: (0, 0, 0)),
        _row_spec,
    ],
    out_specs=[_dis_spec, _half_spec, _half_spec],
    out_shape=[
        jax.ShapeDtypeStruct((N_SEG, 1), jnp.float32),
        jax.ShapeDtypeStruct((N_PAD, DH), jnp.float32),
        jax.ShapeDtypeStruct((N_PAD, DH), jnp.float32),
    ],
)


_p_spec = pl.BlockSpec((NC, NROW_BLK, DH), lambda i: (0, i, 0))


def _comb_g_body(p_ref, dis_ref, g_ref):
    dis = dis_ref[...]
    g_ref[...] = (-dis * dis) * (p_ref[0] + p_ref[1])


_t_comb_g = pl.pallas_call(
    _comb_g_body,
    grid=(GRID_N,),
    in_specs=[_p_spec, _dis_spec],
    out_specs=_half_spec,
    out_shape=jax.ShapeDtypeStruct((N_PAD, DH), jnp.float32),
)


def _final_body(x_ref, p1l, p1h, p2l, p2h, p3l, p3h, p4l, p4h, dis_ref,
                w_ref, b_ref, lng_ref, lnb_ref, o_ref):
    x = x_ref[...]
    dis = dis_ref[...]

    def y(pl_ref, ph_ref):
        return jnp.concatenate(
            [(-dis) * (pl_ref[0] + pl_ref[1]),
             (-dis) * (ph_ref[0] + ph_ref[1])], axis=-1)

    y1 = y(p1l, p1h)
    y2 = y(p2l, p2h)
    y3 = y(p3l, p3h)
    y4 = y(p4l, p4h)
    terms = (x, y1, 2.0 * y2 - x, 4.0 * y3 - 3.0 * y1,
             8.0 * y4 - 8.0 * y2 + x)
    acc = jnp.dot(terms[0], w_ref[0], preferred_element_type=jnp.float32)
    for k in range(1, 5):
        acc = acc + jnp.dot(terms[k], w_ref[k],
                            preferred_element_type=jnp.float32)
    a = acc + b_ref[...]
    ge = 0.5 * a * (1.0 + lax.erf(a * 0.7071067811865476))
    mu = jnp.mean(ge, axis=-1, keepdims=True)
    var = jnp.mean((ge - mu) ** 2, axis=-1, keepdims=True)
    o_ref[...] = (ge - mu) * lax.rsqrt(var + 1e-5) * lng_ref[...] + lnb_ref[...]


_vec_spec = pl.BlockSpec((1, D), lambda i: (0, 0))

_t_final = pl.pallas_call(
    _final_body,
    grid=(GRID_N,),
    in_specs=[
        _row_spec,
        _p_spec, _p_spec, _p_spec, _p_spec,
        _p_spec, _p_spec, _p_spec, _p_spec,
        _dis_spec,
        pl.BlockSpec((5, D, D), lambda i: (0, 0, 0)),
        _vec_spec, _vec_spec, _vec_spec,
    ],
    out_specs=_row_spec,
    out_shape=jax.ShapeDtypeStruct((N_SEG, D), jnp.float32),
)


def kernel(projected_region_features, segment2region_assignment, edge_index,
           cheb_W, cheb_b, ln_g, ln_b):
    pad = N_SEG + (jnp.arange(E_PAD - E, dtype=jnp.int32) % (N_PAD - N_SEG))
    srcr = jnp.concatenate([edge_index[0], pad]).reshape(NB_TOTAL, EB)
    dstr = jnp.concatenate([edge_index[1], pad]).reshape(NB_TOTAL, EB)
    zeros_deg = jnp.zeros((ROWS_ACC, DEG_W), jnp.float32)
    zeros_mv = jnp.zeros((ROWS_ACC, DH), jnp.float32)

    degp = _sc_deg(srcr, zeros_deg)
    raw = _t_matmul(segment2region_assignment, projected_region_features)
    dis, glo, ghi = _t_dis_g(degp, raw)

    p1l = _sc_matvec(glo, srcr, dstr, zeros_mv)
    glo = _t_comb_g(p1l, dis)
    p1h = _sc_matvec(ghi, srcr, dstr, zeros_mv)
    ghi = _t_comb_g(p1h, dis)

    p2l = _sc_matvec(glo, srcr, dstr, zeros_mv)
    glo = _t_comb_g(p2l, dis)
    p2h = _sc_matvec(ghi, srcr, dstr, zeros_mv)
    ghi = _t_comb_g(p2h, dis)

    p3l = _sc_matvec(glo, srcr, dstr, zeros_mv)
    glo = _t_comb_g(p3l, dis)
    p3h = _sc_matvec(ghi, srcr, dstr, zeros_mv)
    ghi = _t_comb_g(p3h, dis)

    p4l = _sc_matvec(glo, srcr, dstr, zeros_mv)
    p4h = _sc_matvec(ghi, srcr, dstr, zeros_mv)

    out = _t_final(raw, p1l, p1h, p2l, p2h, p3l, p3h, p4l, p4h, dis, cheb_W,
                   cheb_b.reshape(1, D), ln_g.reshape(1, D),
                   ln_b.reshape(1, D))
    return (out, raw)


# speedup vs baseline: 1.0868x; 1.0868x over previous
import functools

import jax
import jax.numpy as jnp
from jax import lax
from jax.experimental import pallas as pl
from jax.experimental.pallas import tpu as pltpu
from jax.experimental.pallas import tpu_sc as plsc

NC, NS, LANES = 2, 16, 16
NW = NC * NS

N_SEG = 10000
R_DIM = 1000
D = 128
DH = D // 2
E = 320000
EB = 128
N_PAD = 10240
DUMP = N_PAD - 1
NB_TOTAL = 2560
E_PAD = NB_TOTAL * EB
RPT = NB_TOTAL // NW
KG = 4
NT = RPT // (2 * KG)
ROWS_ACC = N_PAD // NS
DEG_W = 16
NROW_BLK = 1000
GRID_N = N_SEG // NROW_BLK


def _sc_deg_body(src_hbm, zeros_hbm, out_hbm, idx_v, buf_v, sem, acc):
    c = lax.axis_index("c")
    s = lax.axis_index("s")
    wid = c * NS + s

    pltpu.sync_copy(zeros_hbm, acc.at[pl.ds(s * ROWS_ACC, ROWS_ACC)])

    o = jnp.ones((LANES,), jnp.float32)

    def fill_one(i, _):
        buf_v[i, :] = o
        return 0

    lax.fori_loop(0, EB, fill_one, 0)
    pltpu.sync_copy(src_hbm.at[pl.ds(wid * RPT, RPT)], idx_v)
    plsc.subcore_barrier()

    def step(t, _):
        descs = [
            pltpu.async_copy(buf_v, acc.at[idx_v.at[t * 5 + i]], sem,
                             add=True)
            for i in range(5)
        ]
        for d in descs:
            d.wait()
        return 0

    lax.fori_loop(0, RPT // 5, step, 0)
    plsc.subcore_barrier()
    pltpu.sync_copy(acc.at[pl.ds(s * ROWS_ACC, ROWS_ACC)],
                    out_hbm.at[c, pl.ds(s * ROWS_ACC, ROWS_ACC)])


_sc_deg = pl.kernel(
    _sc_deg_body,
    out_type=jax.ShapeDtypeStruct((NC, N_PAD, DEG_W), jnp.float32),
    mesh=plsc.VectorSubcoreMesh(core_axis_name="c", subcore_axis_name="s"),
    scratch_types=[
        pltpu.VMEM((RPT, EB), jnp.int32),
        pltpu.VMEM((EB, DEG_W), jnp.float32),
        pltpu.SemaphoreType.DMA,
        pltpu.VMEM_SHARED((N_PAD, DEG_W), jnp.float32),
    ],
    compiler_params=pltpu.CompilerParams(use_tc_tiling_on_sc=False),
)


def _sc_matvec_body(g_hbm, src_hbm, dst_hbm, zeros_hbm, out_hbm,
                    isrc_v, idst_v, rows_v, gsem, ssem_a, ssem_b, acc):
    c = lax.axis_index("c")
    s = lax.axis_index("s")
    wid = c * NS + s

    pltpu.sync_copy(src_hbm.at[pl.ds(wid * RPT, RPT)], isrc_v)
    pltpu.sync_copy(dst_hbm.at[pl.ds(wid * RPT, RPT)], idst_v)
    pltpu.sync_copy(zeros_hbm, acc.at[pl.ds(s * ROWS_ACC, ROWS_ACC)])
    for b in range(KG):
        pltpu.async_copy(g_hbm.at[pl.ds(0, EB)], rows_v.at[KG + b],
                         ssem_b)
    plsc.subcore_barrier()

    def step(t, _):
        base = t * 2 * KG
        ga = [
            pltpu.async_copy(g_hbm.at[isrc_v.at[base + i]],
                             rows_v.at[i], gsem)
            for i in range(KG)
        ]
        for d in ga:
            d.wait()
        sa = [
            pltpu.async_copy(rows_v.at[i], acc.at[idst_v.at[base + i]],
                             ssem_a, add=True)
            for i in range(KG)
        ]
        for b in range(KG):
            pltpu.make_async_copy(g_hbm.at[pl.ds(0, EB)],
                                  rows_v.at[KG + b], ssem_b).wait()
        gb = [
            pltpu.async_copy(g_hbm.at[isrc_v.at[base + KG + i]],
                             rows_v.at[KG + i], gsem)
            for i in range(KG)
        ]
        for d in gb:
            d.wait()
        for d in sa:
            d.wait()
        for i in range(KG):
            pltpu.async_copy(rows_v.at[KG + i],
                             acc.at[idst_v.at[base + KG + i]],
                             ssem_b, add=True)
        return 0

    lax.fori_loop(0, NT, step, 0)
    for b in range(KG):
        pltpu.make_async_copy(g_hbm.at[pl.ds(0, EB)],
                              rows_v.at[KG + b], ssem_b).wait()
    plsc.subcore_barrier()
    pltpu.sync_copy(acc.at[pl.ds(s * ROWS_ACC, ROWS_ACC)],
                    out_hbm.at[c, pl.ds(s * ROWS_ACC, ROWS_ACC)])


_sc_matvec = pl.kernel(
    _sc_matvec_body,
    out_type=jax.ShapeDtypeStruct((NC, N_PAD, DH), jnp.float32),
    mesh=plsc.VectorSubcoreMesh(core_axis_name="c", subcore_axis_name="s"),
    scratch_types=[
        pltpu.VMEM((RPT, EB), jnp.int32),
        pltpu.VMEM((RPT, EB), jnp.int32),
        pltpu.VMEM((2 * KG, EB, DH), jnp.float32),
        pltpu.SemaphoreType.DMA,
        pltpu.SemaphoreType.DMA,
        pltpu.SemaphoreType.DMA,
        pltpu.VMEM_SHARED((N_PAD, DH), jnp.float32),
    ],
    compiler_params=pltpu.CompilerParams(use_tc_tiling_on_sc=False),
)


_half_spec = pl.BlockSpec((NROW_BLK, DH), lambda i: (i, 0))
_row_spec = pl.BlockSpec((NROW_BLK, D), lambda i: (i, 0))
_dis_spec = pl.BlockSpec((NROW_BLK, 1), lambda i: (i, 0))


def _mm_body(s_ref, p_ref, raw_ref):
    raw_ref[...] = jnp.dot(s_ref[...], p_ref[...],
                           preferred_element_type=jnp.float32)


_t_matmul = pl.pallas_call(
    _mm_body,
    grid=(GRID_N,),
    in_specs=[
        pl.BlockSpec((NROW_BLK, R_DIM), lambda i: (i, 0)),
        pl.BlockSpec((R_DIM, D), lambda i: (0, 0)),
    ],
    out_specs=_row_spec,
    out_shape=jax.ShapeDtypeStruct((N_SEG, D), jnp.float32),
)


def _dis_g_body(p_ref, raw_ref, dis_ref, glo_ref, ghi_ref):
    i = pl.program_id(0)
    d = (p_ref[0, pl.ds(i * NROW_BLK, NROW_BLK), 0:1]
         + p_ref[1, pl.ds(i * NROW_BLK, NROW_BLK), 0:1])
    dis = jnp.where(d > 0.0, lax.rsqrt(jnp.maximum(d, 1.0)), 0.0)
    dis_ref[...] = dis
    g = raw_ref[...] * dis
    glo_ref[...] = g[:, :DH]
    ghi_ref[...] = g[:, DH:]


_t_dis_g = pl.pallas_call(
    _dis_g_body,
    grid=(GRID_N,),
    in_specs=[
        pl.BlockSpec((NC, N_PAD, DEG_W), lambda i: (0, 0, 0)),
        _row_spec,
    ],
    out_specs=[_dis_spec, _half_spec, _half_spec],
    out_shape=[
        jax.ShapeDtypeStruct((N_SEG, 1), jnp.float32),
        jax.ShapeDtypeStruct((N_PAD, DH), jnp.float32),
        jax.ShapeDtypeStruct((N_PAD, DH), jnp.float32),
    ],
)


_p_spec = pl.BlockSpec((NC, NROW_BLK, DH), lambda i: (0, i, 0))


def _comb_g_body(p_ref, dis_ref, g_ref):
    dis = dis_ref[...]
    g_ref[...] = (-dis * dis) * (p_ref[0] + p_ref[1])


_t_comb_g = pl.pallas_call(
    _comb_g_body,
    grid=(GRID_N,),
    in_specs=[_p_spec, _dis_spec],
    out_specs=_half_spec,
    out_shape=jax.ShapeDtypeStruct((N_PAD, DH), jnp.float32),
)


def _final_body(x_ref, p1l, p1h, p2l, p2h, p3l, p3h, p4l, p4h, dis_ref,
                w_ref, b_ref, lng_ref, lnb_ref, o_ref):
    x = x_ref[...]
    dis = dis_ref[...]

    def y(pl_ref, ph_ref):
        return jnp.concatenate(
            [(-dis) * (pl_ref[0] + pl_ref[1]),
             (-dis) * (ph_ref[0] + ph_ref[1])], axis=-1)

    y1 = y(p1l, p1h)
    y2 = y(p2l, p2h)
    y3 = y(p3l, p3h)
    y4 = y(p4l, p4h)
    terms = (x, y1, 2.0 * y2 - x, 4.0 * y3 - 3.0 * y1,
             8.0 * y4 - 8.0 * y2 + x)
    acc = jnp.dot(terms[0], w_ref[0], preferred_element_type=jnp.float32)
    for k in range(1, 5):
        acc = acc + jnp.dot(terms[k], w_ref[k],
                            preferred_element_type=jnp.float32)
    a = acc + b_ref[...]
    ge = 0.5 * a * (1.0 + lax.erf(a * 0.7071067811865476))
    mu = jnp.mean(ge, axis=-1, keepdims=True)
    var = jnp.mean((ge - mu) ** 2, axis=-1, keepdims=True)
    o_ref[...] = (ge - mu) * lax.rsqrt(var + 1e-5) * lng_ref[...] + lnb_ref[...]


_vec_spec = pl.BlockSpec((1, D), lambda i: (0, 0))

_t_final = pl.pallas_call(
    _final_body,
    grid=(GRID_N,),
    in_specs=[
        _row_spec,
        _p_spec, _p_spec, _p_spec, _p_spec,
        _p_spec, _p_spec, _p_spec, _p_spec,
        _dis_spec,
        pl.BlockSpec((5, D, D), lambda i: (0, 0, 0)),
        _vec_spec, _vec_spec, _vec_spec,
    ],
    out_specs=_row_spec,
    out_shape=jax.ShapeDtypeStruct((N_SEG, D), jnp.float32),
)


def kernel(projected_region_features, segment2region_assignment, edge_index,
           cheb_W, cheb_b, ln_g, ln_b):
    pad = N_SEG + (jnp.arange(E_PAD - E, dtype=jnp.int32) % (N_PAD - N_SEG))
    srcr = jnp.concatenate([edge_index[0], pad]).reshape(NB_TOTAL, EB)
    dstr = jnp.concatenate([edge_index[1], pad]).reshape(NB_TOTAL, EB)
    zeros_deg = jnp.zeros((ROWS_ACC, DEG_W), jnp.float32)
    zeros_mv = jnp.zeros((ROWS_ACC, DH), jnp.float32)

    degp = _sc_deg(srcr, zeros_deg)
    raw = _t_matmul(segment2region_assignment, projected_region_features)
    dis, glo, ghi = _t_dis_g(degp, raw)

    p1l = _sc_matvec(glo, srcr, dstr, zeros_mv)
    glo = _t_comb_g(p1l, dis)
    p1h = _sc_matvec(ghi, srcr, dstr, zeros_mv)
    ghi = _t_comb_g(p1h, dis)

    p2l = _sc_matvec(glo, srcr, dstr, zeros_mv)
    glo = _t_comb_g(p2l, dis)
    p2h = _sc_matvec(ghi, srcr, dstr, zeros_mv)
    ghi = _t_comb_g(p2h, dis)

    p3l = _sc_matvec(glo, srcr, dstr, zeros_mv)
    glo = _t_comb_g(p3l, dis)
    p3h = _sc_matvec(ghi, srcr, dstr, zeros_mv)
    ghi = _t_comb_g(p3h, dis)

    p4l = _sc_matvec(glo, srcr, dstr, zeros_mv)
    p4h = _sc_matvec(ghi, srcr, dstr, zeros_mv)

    out = _t_final(raw, p1l, p1h, p2l, p2h, p3l, p3h, p4l, p4h, dis, cheb_W,
                   cheb_b.reshape(1, D), ln_g.reshape(1, D),
                   ln_b.reshape(1, D))
    return (out, raw)
